# Initial kernel scaffold; baseline (speedup 1.0000x reference)
import jax
import jax.numpy as jnp
from jax.experimental import pallas as pl


def kernel(src_tokens, tgt_tokens, src_table, tgt_table):
    raise NotImplementedError("write your pallas kernel here")


# speedup vs baseline: 1.3425x; 1.3425x over previous
import dataclasses
import functools

import jax
import jax.numpy as jnp
from jax import lax
from jax.experimental import pallas as pl
from jax.experimental.pallas import tpu as pltpu
from jax.experimental.pallas import tpu_sc as plsc

DIM = 32
W = 128
LANES = 16


def _zero_padding_rows(idx_ref, out_ref):
    zeros = jnp.zeros((LANES,), jnp.float32)
    for g in range(W // LANES):
        v = idx_ref.at[0][pl.ds(g * LANES, LANES)]
        is_pad = v == 0

        @pl.when(jnp.any(is_pad))
        def _():
            rows = jnp.arange(LANES, dtype=jnp.int32) + (g * LANES)
            for c in range(DIM):
                cols = jnp.full((LANES,), c, jnp.int32)
                plsc.store_scatter(out_ref, [rows, cols], zeros, mask=is_pad)


def _make_kernel(n_idx):
    mesh = plsc.VectorSubcoreMesh(core_axis_name="c", subcore_axis_name="s")
    out_sds = jax.ShapeDtypeStruct((n_idx, DIM), jnp.float32)

    cp = pltpu.CompilerParams()
    fields = pltpu.CompilerParams.__dataclass_fields__
    if "needs_layout_passes" in fields:
        cp = dataclasses.replace(cp, needs_layout_passes=False)
    if "use_tc_tiling_on_sc" in fields:
        cp = dataclasses.replace(cp, use_tc_tiling_on_sc=False)

    @functools.partial(
        pl.kernel,
        out_type=(out_sds, out_sds),
        mesh=mesh,
        compiler_params=cp,
    )
    def emb_kernel(src_table_hbm, tgt_table_hbm, src_idx_hbm, tgt_idx_hbm,
                   src_out_hbm, tgt_out_hbm):
        def body(src_i, tgt_i, src_o, tgt_o):
            pltpu.sync_copy(src_table_hbm.at[src_i.at[0]], src_o)
            pltpu.sync_copy(tgt_table_hbm.at[tgt_i.at[0]], tgt_o)
            _zero_padding_rows(src_i, src_o)
            _zero_padding_rows(tgt_i, tgt_o)

        idx_spec = pl.BlockSpec((1, W), lambda i: (0, i))
        out_spec = pl.BlockSpec((W, DIM), lambda i: (i, 0))
        pltpu.emit_pipeline(
            body,
            grid=(n_idx // W,),
            in_specs=[idx_spec, idx_spec],
            out_specs=[out_spec, out_spec],
            core_axis_name=("c", "s"),
            dimension_semantics=(pltpu.PARALLEL,),
        )(src_idx_hbm, tgt_idx_hbm, src_out_hbm, tgt_out_hbm)

    return emb_kernel


def kernel(src_tokens, tgt_tokens, src_table, tgt_table):
    b, l = src_tokens.shape
    n_idx = b * l
    src_idx = src_tokens.reshape(1, n_idx).astype(jnp.int32)
    tgt_idx = tgt_tokens.reshape(1, n_idx).astype(jnp.int32)
    emb = _make_kernel(n_idx)
    src_out, tgt_out = emb(src_table, tgt_table, src_idx, tgt_idx)
    return (src_out.reshape(b, l, DIM), tgt_out.reshape(b, l, DIM))


# speedup vs baseline: 1.4362x; 1.0698x over previous
import dataclasses
import functools

import jax
import jax.numpy as jnp
from jax.experimental import pallas as pl
from jax.experimental.pallas import tpu as pltpu
from jax.experimental.pallas import tpu_sc as plsc

DIM = 32
W = 128
K = 4
LANES = 16


def _zero_padding_rows(idx_row, out_ref, row_base):
    zeros = jnp.zeros((LANES,), jnp.float32)
    for g in range(W // LANES):
        v = idx_row[pl.ds(g * LANES, LANES)]
        is_pad = v == 0

        @pl.when(jnp.any(is_pad))
        def _():
            rows = jnp.arange(LANES, dtype=jnp.int32) + (row_base + g * LANES)
            for c in range(DIM):
                cols = jnp.full((LANES,), c, jnp.int32)
                plsc.store_scatter(out_ref, [rows, cols], zeros, mask=is_pad)


def _make_kernel(n_idx):
    mesh = plsc.VectorSubcoreMesh(core_axis_name="c", subcore_axis_name="s")
    out_sds = jax.ShapeDtypeStruct((n_idx, DIM), jnp.float32)

    cp = pltpu.CompilerParams()
    fields = pltpu.CompilerParams.__dataclass_fields__
    if "needs_layout_passes" in fields:
        cp = dataclasses.replace(cp, needs_layout_passes=False)
    if "use_tc_tiling_on_sc" in fields:
        cp = dataclasses.replace(cp, use_tc_tiling_on_sc=False)

    @functools.partial(
        pl.kernel,
        out_type=(out_sds, out_sds),
        mesh=mesh,
        compiler_params=cp,
        scratch_types=[pltpu.SemaphoreType.DMA],
    )
    def emb_kernel(src_table_hbm, tgt_table_hbm, src_idx_hbm, tgt_idx_hbm,
                   src_out_hbm, tgt_out_hbm, sem):
        def body(src_i, tgt_i, src_o, tgt_o):
            copies = []
            for j in range(K):
                copies.append(pltpu.async_copy(
                    src_table_hbm.at[src_i.at[j]],
                    src_o.at[pl.ds(j * W, W)], sem))
                copies.append(pltpu.async_copy(
                    tgt_table_hbm.at[tgt_i.at[j]],
                    tgt_o.at[pl.ds(j * W, W)], sem))
            for cp_ in copies:
                cp_.wait()
            for j in range(K):
                _zero_padding_rows(src_i.at[j], src_o, j * W)
                _zero_padding_rows(tgt_i.at[j], tgt_o, j * W)

        idx_spec = pl.BlockSpec((K, W), lambda i: (i, 0))
        out_spec = pl.BlockSpec((K * W, DIM), lambda i: (i, 0))
        pltpu.emit_pipeline(
            body,
            grid=(n_idx // (K * W),),
            in_specs=[idx_spec, idx_spec],
            out_specs=[out_spec, out_spec],
            core_axis_name=("c", "s"),
            dimension_semantics=(pltpu.PARALLEL,),
        )(src_idx_hbm, tgt_idx_hbm, src_out_hbm, tgt_out_hbm)

    return emb_kernel


def kernel(src_tokens, tgt_tokens, src_table, tgt_table):
    b, l = src_tokens.shape
    n_idx = b * l
    src_idx = src_tokens.reshape(n_idx // W, W).astype(jnp.int32)
    tgt_idx = tgt_tokens.reshape(n_idx // W, W).astype(jnp.int32)
    emb = _make_kernel(n_idx)
    src_out, tgt_out = emb(src_table, tgt_table, src_idx, tgt_idx)
    return (src_out.reshape(b, l, DIM), tgt_out.reshape(b, l, DIM))
